# baseline (speedup 1.0000x reference)
import math

import jax
import jax.numpy as jnp
from jax.experimental import pallas as pl
from jax.experimental.pallas import tpu as pltpu

B = 2
T = 2048
D_MODEL = 1024
SKILL_DIM = 128
MAX_SKILLS = 4096
TOP_K = 8

NJ = 8
NT = 8
DJ = D_MODEL // NJ
TT = T // NT


def _fused_kernel(h_last_ref, embeds_ref, wq_ref, bq_ref, wk_ref, bk_ref,
                  wkv_ref, bkv_ref, wo_ref, h_exec_ref, skill_ref,
                  v_scr, row_scr):
    j = pl.program_id(0)
    t = pl.program_id(1)

    @pl.when(jnp.logical_and(j == 0, t == 0))
    def _retrieve():
        q = h_last_ref[...] @ wq_ref[...] + bq_ref[...]
        keys = embeds_ref[...] @ wk_ref[...] + bk_ref[...]
        s = (q @ keys.T) * (1.0 / math.sqrt(SKILL_DIM))

        iota = jax.lax.broadcasted_iota(jnp.int32, (B, MAX_SKILLS), 1)
        work = s
        sel = jnp.zeros((B, MAX_SKILLS), dtype=jnp.bool_)
        for _ in range(TOP_K):
            m = jnp.max(work, axis=1, keepdims=True)
            cand = jnp.where(work == m, iota, MAX_SKILLS)
            amin = jnp.min(cand, axis=1, keepdims=True)
            pick = iota == amin
            sel = jnp.logical_or(sel, pick)
            work = jnp.where(pick, -jnp.inf, work)

        logits = jnp.where(sel, s, -jnp.inf)
        mx = jnp.max(logits, axis=1, keepdims=True)
        e = jnp.exp(logits - mx)
        w = e / jnp.sum(e, axis=1, keepdims=True)
        skill = w @ embeds_ref[...]

        skill_ref[...] = skill
        v_scr[...] = skill @ wkv_ref[...] + bkv_ref[...]

    @pl.when(t == 0)
    def _project():
        row_scr[...] = v_scr[...] @ wo_ref[...]

    h_exec_ref[...] = jnp.broadcast_to(row_scr[...][:, None, :], (B, TT, DJ))


def kernel(h, embeds, Wq_r, bq_r, Wk_r, bk_r, Wc, bc, gate, Wkv, bkv, Wq_a, Wo):
    h_exec, skill = pl.pallas_call(
        _fused_kernel,
        grid=(NJ, NT),
        in_specs=[
            pl.BlockSpec((B, D_MODEL), lambda j, t: (0, 0)),
            pl.BlockSpec((MAX_SKILLS, SKILL_DIM), lambda j, t: (0, 0)),
            pl.BlockSpec((D_MODEL, SKILL_DIM), lambda j, t: (0, 0)),
            pl.BlockSpec((1, SKILL_DIM), lambda j, t: (0, 0)),
            pl.BlockSpec((SKILL_DIM, SKILL_DIM), lambda j, t: (0, 0)),
            pl.BlockSpec((1, SKILL_DIM), lambda j, t: (0, 0)),
            pl.BlockSpec((SKILL_DIM, D_MODEL), lambda j, t: (0, 1)),
            pl.BlockSpec((1, D_MODEL), lambda j, t: (0, 1)),
            pl.BlockSpec((D_MODEL, DJ), lambda j, t: (0, j)),
        ],
        out_specs=[
            pl.BlockSpec((B, TT, DJ), lambda j, t: (0, t, j)),
            pl.BlockSpec((B, SKILL_DIM), lambda j, t: (0, 0)),
        ],
        out_shape=(
            jax.ShapeDtypeStruct((B, T, D_MODEL), jnp.float32),
            jax.ShapeDtypeStruct((B, SKILL_DIM), jnp.float32),
        ),
        scratch_shapes=[
            pltpu.VMEM((B, D_MODEL), jnp.float32),
            pltpu.VMEM((B, DJ), jnp.float32),
        ],
    )(h[:, -1], embeds, Wq_r, bq_r.reshape(1, SKILL_DIM), Wk_r,
      bk_r.reshape(1, SKILL_DIM), Wkv, bkv.reshape(1, 2 * D_MODEL), Wo)
    return (h_exec, skill)


# speedup vs baseline: 1.0236x; 1.0236x over previous
import math

import jax
import jax.numpy as jnp
from jax import lax
from jax.experimental import pallas as pl
from jax.experimental.pallas import tpu as pltpu
from jax.experimental.pallas import tpu_sc as plsc

B = 2
T = 2048
D_MODEL = 1024
SKILL_DIM = 128
MAX_SKILLS = 4096
TOP_K = 8

NC = 2
NS = 16
L = 16
CHUNK = MAX_SKILLS // NS
NV = CHUNK // L
NEG = float("-inf")


def _scores_kernel(h_last_ref, wq_ref, bq_ref, embeds_ref, wk_ref, bk_ref,
                   scores_ref):
    q = h_last_ref[...] @ wq_ref[...] + bq_ref[...]
    keys = embeds_ref[...] @ wk_ref[...] + bk_ref[...]
    scores_ref[...] = (q @ keys.T) * (1.0 / math.sqrt(SKILL_DIM))


def _xlane(x, sh):
    perm = (lax.iota(jnp.int32, L) ^ sh).reshape(L, 1)
    return lax.gather(
        x, perm,
        dimension_numbers=lax.GatherDimensionNumbers(
            offset_dims=(), collapsed_slice_dims=(0,), start_index_map=(0,)),
        slice_sizes=(1,),
        unique_indices=True,
        mode=lax.GatherScatterMode.PROMISE_IN_BOUNDS)


def _bcast_max(x):
    for sh in (8, 4, 2, 1):
        x = jnp.maximum(x, _xlane(x, sh))
    return x


def _bcast_min(x):
    for sh in (8, 4, 2, 1):
        x = jnp.minimum(x, _xlane(x, sh))
    return x


def _bcast_sum(x):
    for sh in (8, 4, 2, 1):
        x = x + _xlane(x, sh)
    return x


def _sc_topk_body(scores_hbm, embeds_hbm, skill_hbm, cand_vals_hbm,
                  cand_idx_hbm, chunk_v, stage_v, stage_i, merge_v, merge_i,
                  rows_v, idx_v, skill_row, sem):
    c = lax.axis_index("c")
    s = lax.axis_index("s")
    w = c * NS + s

    pltpu.sync_copy(scores_hbm.at[pl.ds(w * CHUNK, CHUNK)], chunk_v)
    vs = [chunk_v[pl.ds(i * L, L)] for i in range(NV)]
    base = s * CHUNK
    gidx = [lax.iota(jnp.int32, L) + (base + i * L) for i in range(NV)]

    val_vec = jnp.full((L,), NEG, dtype=jnp.float32)
    idx_vec = jnp.zeros((L,), dtype=jnp.int32)
    lane = lax.iota(jnp.int32, L)
    for k in range(TOP_K):
        m = vs[0]
        for i in range(1, NV):
            m = jnp.maximum(m, vs[i])
        mx = _bcast_max(m)
        cmin = jnp.where(vs[0] == mx, gidx[0], MAX_SKILLS)
        for i in range(1, NV):
            cmin = jnp.minimum(cmin, jnp.where(vs[i] == mx, gidx[i], MAX_SKILLS))
        amin = _bcast_min(cmin)
        val_vec = jnp.where(lane == k, mx, val_vec)
        idx_vec = jnp.where(lane == k, amin, idx_vec)
        vs = [jnp.where(gidx[i] == amin, NEG, vs[i]) for i in range(NV)]

    stage_v[...] = val_vec
    stage_i[...] = idx_vec
    pltpu.sync_copy(stage_v, cand_vals_hbm.at[pl.ds(w * L, L)])
    pltpu.sync_copy(stage_i, cand_idx_hbm.at[pl.ds(w * L, L)])
    plsc.subcore_barrier()

    @pl.when(s == 0)
    def _merge():
        pltpu.sync_copy(cand_vals_hbm.at[pl.ds(c * NS * L, NS * L)], merge_v)
        pltpu.sync_copy(cand_idx_hbm.at[pl.ds(c * NS * L, NS * L)], merge_i)
        cur = [merge_v[pl.ds(i * L, L)] for i in range(NS)]
        mis = [merge_i[pl.ds(i * L, L)] for i in range(NS)]

        tval = jnp.full((L,), NEG, dtype=jnp.float32)
        tidx = jnp.zeros((L,), dtype=jnp.int32)
        for k in range(TOP_K):
            m = cur[0]
            for i in range(1, NS):
                m = jnp.maximum(m, cur[i])
            mx = _bcast_max(m)
            cmin = jnp.where(cur[0] == mx, mis[0], MAX_SKILLS)
            for i in range(1, NS):
                cmin = jnp.minimum(cmin, jnp.where(cur[i] == mx, mis[i], MAX_SKILLS))
            amin = _bcast_min(cmin)
            tval = jnp.where(lane == k, mx, tval)
            tidx = jnp.where(lane == k, amin, tidx)
            cur = [jnp.where(mis[i] == amin, NEG, cur[i]) for i in range(NS)]

        mx0 = _bcast_max(tval)
        ex = jnp.exp(tval - mx0)
        wn = ex / _bcast_sum(ex)
        idx_v[...] = tidx

        pltpu.async_copy(embeds_hbm.at[idx_v], rows_v, sem).wait()
        for t in range(SKILL_DIM // L):
            acc = jnp.zeros((L,), dtype=jnp.float32)
            for r in range(TOP_K):
                acc = acc + wn[r] * rows_v[r, pl.ds(t * L, L)]
            skill_row[pl.ds(t * L, L)] = acc
        pltpu.sync_copy(skill_row, skill_hbm.at[pl.ds(c * SKILL_DIM, SKILL_DIM)])


def _sc_topk(scores, embeds):
    mesh = plsc.VectorSubcoreMesh(core_axis_name="c", subcore_axis_name="s",
                                  num_cores=NC, num_subcores=NS)
    skill_flat, _cv, _ci = pl.kernel(
        _sc_topk_body,
        out_type=(
            jax.ShapeDtypeStruct((B * SKILL_DIM,), jnp.float32),
            jax.ShapeDtypeStruct((B * NS * L,), jnp.float32),
            jax.ShapeDtypeStruct((B * NS * L,), jnp.int32),
        ),
        mesh=mesh,
        scratch_types=[
            pltpu.VMEM((CHUNK,), jnp.float32),
            pltpu.VMEM((L,), jnp.float32),
            pltpu.VMEM((L,), jnp.int32),
            pltpu.VMEM((NS * L,), jnp.float32),
            pltpu.VMEM((NS * L,), jnp.int32),
            pltpu.VMEM((L, SKILL_DIM), jnp.float32),
            pltpu.VMEM((L,), jnp.int32),
            pltpu.VMEM((SKILL_DIM,), jnp.float32),
            pltpu.SemaphoreType.DMA,
        ],
    )(scores.reshape(-1), embeds)
    return skill_flat.reshape(B, SKILL_DIM)


def _project_kernel(skill_ref, wkv_ref, bkv_ref, wo_ref, h_exec_ref):
    v = skill_ref[...] @ wkv_ref[...] + bkv_ref[...]
    row = v @ wo_ref[...]
    h_exec_ref[...] = jnp.broadcast_to(row[:, None, :], (B, T, D_MODEL))


def kernel(h, embeds, Wq_r, bq_r, Wk_r, bk_r, Wc, bc, gate, Wkv, bkv, Wq_a, Wo):
    scores = pl.pallas_call(
        _scores_kernel,
        grid=(1,),
        in_specs=[
            pl.BlockSpec((B, D_MODEL), lambda i: (0, 0)),
            pl.BlockSpec((D_MODEL, SKILL_DIM), lambda i: (0, 0)),
            pl.BlockSpec((1, SKILL_DIM), lambda i: (0, 0)),
            pl.BlockSpec((MAX_SKILLS, SKILL_DIM), lambda i: (0, 0)),
            pl.BlockSpec((SKILL_DIM, SKILL_DIM), lambda i: (0, 0)),
            pl.BlockSpec((1, SKILL_DIM), lambda i: (0, 0)),
        ],
        out_specs=pl.BlockSpec((B, MAX_SKILLS), lambda i: (0, 0)),
        out_shape=jax.ShapeDtypeStruct((B, MAX_SKILLS), jnp.float32),
    )(h[:, -1], Wq_r, bq_r.reshape(1, SKILL_DIM), embeds, Wk_r,
      bk_r.reshape(1, SKILL_DIM))

    skill = _sc_topk(scores, embeds)

    h_exec = pl.pallas_call(
        _project_kernel,
        grid=(1,),
        in_specs=[
            pl.BlockSpec((B, SKILL_DIM), lambda i: (0, 0)),
            pl.BlockSpec((SKILL_DIM, D_MODEL), lambda i: (0, 1)),
            pl.BlockSpec((1, D_MODEL), lambda i: (0, 1)),
            pl.BlockSpec((D_MODEL, D_MODEL), lambda i: (0, 0)),
        ],
        out_specs=pl.BlockSpec((B, T, D_MODEL), lambda i: (0, 0, 0)),
        out_shape=jax.ShapeDtypeStruct((B, T, D_MODEL), jnp.float32),
    )(skill, Wkv, bkv.reshape(1, 2 * D_MODEL), Wo)
    return (h_exec, skill)
